# baseline (device time: 50347 ns/iter reference)
import functools

import jax
import jax.numpy as jnp
from jax import lax
from jax.experimental import pallas as pl
from jax.experimental.pallas import tpu as pltpu

N_DEV = 4
B_PER = 2
SQ = 128
HG = 4
DH = 64
D_MODEL = 512
D_HEADS = HG * DH


def _body(x_ref, w_ref, k_ref, v_ref, out_ref, comm_ref, send_sems, recv_sems):
    my = lax.axis_index("i")
    right = lax.rem(my + 1, N_DEV)
    left = lax.rem(my + N_DEV - 1, N_DEV)

    barrier_sem = pltpu.get_barrier_semaphore()
    for nbr in (left, right):
        pl.semaphore_signal(
            barrier_sem, inc=1,
            device_id=(nbr,), device_id_type=pl.DeviceIdType.MESH,
        )
    pl.semaphore_wait(barrier_sem, 2)

    comm_ref[0] = w_ref[...]
    xf = x_ref[...]

    acc = None
    for h in range(N_DEV):
        if h < N_DEV - 1:
            rdma = pltpu.make_async_remote_copy(
                src_ref=comm_ref.at[h],
                dst_ref=comm_ref.at[h + 1],
                send_sem=send_sems.at[h],
                recv_sem=recv_sems.at[h + 1],
                device_id=(right,),
                device_id_type=pl.DeviceIdType.MESH,
            )
            rdma.start()

        wq_g = comm_ref[h, 0:D_MODEL, :]
        wot_g = comm_ref[h, D_MODEL:D_MODEL + D_MODEL, :]
        q = jnp.dot(xf, wq_g, preferred_element_type=jnp.float32)
        ctx_rows = []
        for b in range(B_PER):
            ctx_heads = []
            for j in range(HG):
                qbh = q[b * SQ:(b + 1) * SQ, j * DH:(j + 1) * DH]
                kt = k_ref[b, h * HG + j]
                s = jnp.dot(qbh, kt, preferred_element_type=jnp.float32) * 0.125
                s = s - jnp.max(s, axis=1, keepdims=True)
                e = jnp.exp(s)
                w = e / jnp.sum(e, axis=1, keepdims=True)
                vb = v_ref[b, h * HG + j]
                ctx_heads.append(jnp.dot(w, vb, preferred_element_type=jnp.float32))
            ctx_rows.append(jnp.concatenate(ctx_heads, axis=1))
        ctx = jnp.concatenate(ctx_rows, axis=0)
        part = lax.dot_general(
            ctx, wot_g, (((1,), (1,)), ((), ())),
            preferred_element_type=jnp.float32,
        )
        acc = part if acc is None else acc + part

        if h < N_DEV - 1:
            rdma.wait()

    out_ref[...] = acc


def kernel(x, Wq, K_ext, V_ext, Wo):
    my = lax.axis_index("i")

    groups = jnp.mod(my - jnp.arange(N_DEV), N_DEV)
    head_idx = (groups[:, None] * HG + jnp.arange(HG)[None, :]).reshape(-1)
    kb = lax.dynamic_slice_in_dim(K_ext, my * B_PER, B_PER, axis=0)
    vb = lax.dynamic_slice_in_dim(V_ext, my * B_PER, B_PER, axis=0)
    kr = jnp.take(kb, head_idx, axis=2)
    vr = jnp.take(vb, head_idx, axis=2)
    kt = jnp.transpose(kr, (0, 2, 3, 1))
    vt = jnp.transpose(vr, (0, 2, 1, 3))

    xf = x.reshape(B_PER * SQ, D_MODEL)
    packed = jnp.concatenate([Wq, Wo.T], axis=0)

    out = pl.pallas_call(
        _body,
        out_shape=jax.ShapeDtypeStruct((B_PER * SQ, D_MODEL), jnp.float32),
        in_specs=[
            pl.BlockSpec(memory_space=pltpu.VMEM),
            pl.BlockSpec(memory_space=pltpu.VMEM),
            pl.BlockSpec(memory_space=pltpu.VMEM),
            pl.BlockSpec(memory_space=pltpu.VMEM),
        ],
        out_specs=pl.BlockSpec(memory_space=pltpu.VMEM),
        scratch_shapes=[
            pltpu.VMEM((N_DEV, 2 * D_MODEL, D_HEADS), jnp.float32),
            pltpu.SemaphoreType.DMA((N_DEV,)),
            pltpu.SemaphoreType.DMA((N_DEV,)),
        ],
        compiler_params=pltpu.CompilerParams(collective_id=0),
    )(xf, packed, kt, vt)

    return out.reshape(B_PER, SQ, D_MODEL)


# device time: 31557 ns/iter; 1.5954x vs baseline; 1.5954x over previous
import jax
import jax.numpy as jnp
from jax import lax
from jax.experimental import pallas as pl
from jax.experimental.pallas import tpu as pltpu

N_DEV = 4
B_PER = 2
SQ = 128
HG = 4
DH = 64
D_MODEL = 512
D_HEADS = HG * DH


def _group_contrib(xf, wq_g, wot_g, k_ref, v_ref, slot):
    q = jnp.dot(xf, wq_g, preferred_element_type=jnp.float32)
    ctx_rows = []
    for b in range(B_PER):
        ctx_heads = []
        for j in range(HG):
            qbh = q[b * SQ:(b + 1) * SQ, j * DH:(j + 1) * DH]
            kt = k_ref[b, slot * HG + j]
            s = jnp.dot(qbh, kt, preferred_element_type=jnp.float32) * 0.125
            s = s - jnp.max(s, axis=1, keepdims=True)
            e = jnp.exp(s)
            w = e / jnp.sum(e, axis=1, keepdims=True)
            vb = v_ref[b, slot * HG + j]
            ctx_heads.append(jnp.dot(w, vb, preferred_element_type=jnp.float32))
        ctx_rows.append(jnp.concatenate(ctx_heads, axis=1))
    ctx = jnp.concatenate(ctx_rows, axis=0)
    return lax.dot_general(
        ctx, wot_g, (((1,), (1,)), ((), ())),
        preferred_element_type=jnp.float32,
    )


def _body(x_ref, w_ref, k_ref, v_ref, out_ref, comm_ref, send_sems, recv_sems):
    my = lax.axis_index("i")
    right = lax.rem(my + 1, N_DEV)
    left = lax.rem(my + N_DEV - 1, N_DEV)

    barrier_sem = pltpu.get_barrier_semaphore()
    for nbr in (left, right):
        pl.semaphore_signal(
            barrier_sem, inc=1,
            device_id=(nbr,), device_id_type=pl.DeviceIdType.MESH,
        )
    pl.semaphore_wait(barrier_sem, 2)

    comm_ref[pl.ds(0, 2)] = w_ref[...]
    xf = x_ref[...]

    cw0 = pltpu.make_async_remote_copy(
        src_ref=comm_ref.at[pl.ds(0, 2)], dst_ref=comm_ref.at[pl.ds(2, 2)],
        send_sem=send_sems.at[0], recv_sem=recv_sems.at[0],
        device_id=(right,), device_id_type=pl.DeviceIdType.MESH,
    )
    ccw0 = pltpu.make_async_remote_copy(
        src_ref=comm_ref.at[pl.ds(0, 2)], dst_ref=comm_ref.at[pl.ds(4, 2)],
        send_sem=send_sems.at[1], recv_sem=recv_sems.at[1],
        device_id=(left,), device_id_type=pl.DeviceIdType.MESH,
    )
    cw0.start()
    ccw0.start()

    acc = _group_contrib(xf, comm_ref[0], comm_ref[1], k_ref, v_ref, 0)

    cw0.wait_recv()
    ccw0.wait_recv()

    cw1 = pltpu.make_async_remote_copy(
        src_ref=comm_ref.at[2], dst_ref=comm_ref.at[6],
        send_sem=send_sems.at[2], recv_sem=recv_sems.at[2],
        device_id=(right,), device_id_type=pl.DeviceIdType.MESH,
    )
    ccw1 = pltpu.make_async_remote_copy(
        src_ref=comm_ref.at[5], dst_ref=comm_ref.at[7],
        send_sem=send_sems.at[3], recv_sem=recv_sems.at[3],
        device_id=(left,), device_id_type=pl.DeviceIdType.MESH,
    )
    cw1.start()
    ccw1.start()

    acc = acc + _group_contrib(xf, comm_ref[2], comm_ref[3], k_ref, v_ref, 1)
    acc = acc + _group_contrib(xf, comm_ref[4], comm_ref[5], k_ref, v_ref, 2)

    cw1.wait_recv()
    ccw1.wait_recv()
    acc = acc + _group_contrib(xf, comm_ref[6], comm_ref[7], k_ref, v_ref, 3)

    cw0.wait_send()
    ccw0.wait_send()
    cw1.wait_send()
    ccw1.wait_send()

    out_ref[...] = acc


def kernel(x, Wq, K_ext, V_ext, Wo):
    my = lax.axis_index("i")

    groups = jnp.mod(my + jnp.array([0, -1, 1, 2]), N_DEV)
    head_idx = (groups[:, None] * HG + jnp.arange(HG)[None, :]).reshape(-1)
    kb = lax.dynamic_slice_in_dim(K_ext, my * B_PER, B_PER, axis=0)
    vb = lax.dynamic_slice_in_dim(V_ext, my * B_PER, B_PER, axis=0)
    kr = jnp.take(kb, head_idx, axis=2)
    vr = jnp.take(vb, head_idx, axis=2)
    kt = jnp.transpose(kr, (0, 2, 3, 1))
    vt = jnp.transpose(vr, (0, 2, 1, 3))

    xf = x.reshape(B_PER * SQ, D_MODEL)
    packed = jnp.stack([Wq, Wo.T])

    out = pl.pallas_call(
        _body,
        out_shape=jax.ShapeDtypeStruct((B_PER * SQ, D_MODEL), jnp.float32),
        in_specs=[
            pl.BlockSpec(memory_space=pltpu.VMEM),
            pl.BlockSpec(memory_space=pltpu.VMEM),
            pl.BlockSpec(memory_space=pltpu.VMEM),
            pl.BlockSpec(memory_space=pltpu.VMEM),
        ],
        out_specs=pl.BlockSpec(memory_space=pltpu.VMEM),
        scratch_shapes=[
            pltpu.VMEM((2 * N_DEV, D_MODEL, D_HEADS), jnp.float32),
            pltpu.SemaphoreType.DMA((4,)),
            pltpu.SemaphoreType.DMA((4,)),
        ],
        compiler_params=pltpu.CompilerParams(collective_id=0),
    )(xf, packed, kt, vt)

    return out.reshape(B_PER, SQ, D_MODEL)
